# Initial kernel scaffold; baseline (speedup 1.0000x reference)
import jax
import jax.numpy as jnp
from jax.experimental import pallas as pl


def kernel(data, w_router, w1, w2, w3):
    raise NotImplementedError("write your pallas kernel here")


# speedup vs baseline: 1.3525x; 1.3525x over previous
import functools

import jax
import jax.numpy as jnp
from jax import lax
from jax.experimental import pallas as pl
from jax.experimental.pallas import tpu as pltpu
from jax.experimental.pallas import tpu_sc as plsc

_T = 2048
_D = 2048
_DFF = 4096
_E = 8
_BLK = 256
_NB = 24
_PAD = _NB * _BLK
_FT = 512
_NF = _DFF // _FT

_NC = 2
_NS = 16
_NW = _NC * _NS
_TPW = _T // _NW
_RCH = 16


def _router_body(data_ref, wr_ref, logits_ref, p1_ref, p2_ref, g1_ref, g2_ref,
                 bem_ref, nv_ref):
    x = data_ref[...]
    logits = jnp.dot(x, wr_ref[...], preferred_element_type=jnp.float32)
    logits_ref[...] = logits
    m = jnp.max(logits, axis=-1, keepdims=True)
    ex = jnp.exp(logits - m)
    probs = ex / jnp.sum(ex, axis=-1, keepdims=True)
    eidx = lax.broadcasted_iota(jnp.int32, (_T, _E), 1)
    v1 = jnp.max(probs, axis=-1, keepdims=True)
    i1 = jnp.min(jnp.where(probs == v1, eidx, _E), axis=-1, keepdims=True)
    pm = jnp.where(eidx == i1, -1.0, probs)
    v2 = jnp.max(pm, axis=-1, keepdims=True)
    i2 = jnp.min(jnp.where(pm == v2, eidx, _E), axis=-1, keepdims=True)
    s = v1 + v2
    g1 = v1 / s
    g2 = v2 / s
    onehot = jnp.logical_or(eidx == i1, eidx == i2).astype(jnp.float32)
    rr = lax.broadcasted_iota(jnp.int32, (_T, _T), 0)
    cc = lax.broadcasted_iota(jnp.int32, (_T, _T), 1)
    tri = (cc <= rr).astype(jnp.float32)
    cum = jnp.dot(tri, onehot, preferred_element_type=jnp.float32)
    excl = cum - onehot
    cnt = cum[_T - 1:_T, :]
    padded = jnp.floor((cnt + (_BLK - 1)) / _BLK) * _BLK
    re_ = lax.broadcasted_iota(jnp.int32, (_E, _E), 0)
    ce_ = lax.broadcasted_iota(jnp.int32, (_E, _E), 1)
    sut = (re_ < ce_).astype(jnp.float32)
    off = jnp.dot(padded, sut, preferred_element_type=jnp.float32)
    r1 = jnp.sum(jnp.where(eidx == i1, excl, 0.0), axis=-1, keepdims=True)
    o1 = jnp.sum(jnp.where(eidx == i1, off, 0.0), axis=-1, keepdims=True)
    r2 = jnp.sum(jnp.where(eidx == i2, excl, 0.0), axis=-1, keepdims=True)
    o2 = jnp.sum(jnp.where(eidx == i2, off, 0.0), axis=-1, keepdims=True)
    p1_ref[...] = (o1 + r1).astype(jnp.int32)
    p2_ref[...] = (o2 + r2).astype(jnp.int32)
    g1_ref[...] = jnp.broadcast_to(g1, (_T, 128))
    g2_ref[...] = jnp.broadcast_to(g2, (_T, 128))
    bstart = (lax.broadcasted_iota(jnp.int32, (_NB, _E), 0) * _BLK).astype(
        jnp.float32)
    offb = jnp.broadcast_to(off, (_NB, _E))
    eb = jnp.sum((offb <= bstart).astype(jnp.int32), axis=-1,
                 keepdims=True) - 1
    end = jnp.broadcast_to(off + cnt, (_NB, _E))
    enb = lax.broadcasted_iota(jnp.int32, (_NB, _E), 1)
    endb = jnp.sum(jnp.where(enb == eb, end, 0.0), axis=-1, keepdims=True)
    nv = jnp.clip(endb - bstart[:, :1], 0.0, float(_BLK))
    bem_ref[...] = eb
    nv_ref[...] = nv.astype(jnp.int32)


def _router(data, w_router):
    return pl.pallas_call(
        _router_body,
        out_shape=[
            jax.ShapeDtypeStruct((_T, _E), jnp.float32),
            jax.ShapeDtypeStruct((_T, 1), jnp.int32),
            jax.ShapeDtypeStruct((_T, 1), jnp.int32),
            jax.ShapeDtypeStruct((_T, 128), jnp.float32),
            jax.ShapeDtypeStruct((_T, 128), jnp.float32),
            jax.ShapeDtypeStruct((_NB, 1), jnp.int32),
            jax.ShapeDtypeStruct((_NB, 1), jnp.int32),
        ],
    )(data, w_router)


def _dispatch_body(data_hbm, p1_hbm, p2_hbm, g1_hbm, g2_hbm, xs_hbm, gs_hbm,
                   p1_v, p2_v, gbuf, rbuf, sem):
    wid = lax.axis_index("s") * _NC + lax.axis_index("c")
    base = wid * _TPW
    pltpu.sync_copy(p1_hbm.at[pl.ds(base, _TPW)], p1_v)
    pltpu.sync_copy(p2_hbm.at[pl.ds(base, _TPW)], p2_v)
    pltpu.sync_copy(g1_hbm.at[pl.ds(base, _TPW)], gbuf)
    pltpu.async_copy(gbuf, gs_hbm.at[p1_v], sem).wait()
    pltpu.sync_copy(g2_hbm.at[pl.ds(base, _TPW)], gbuf)
    pltpu.async_copy(gbuf, gs_hbm.at[p2_v], sem).wait()
    for c in range(_TPW // _RCH):
        pltpu.sync_copy(data_hbm.at[pl.ds(base + c * _RCH, _RCH)], rbuf)
        i1 = p1_v[pl.ds(c * _RCH, _RCH)]
        i2 = p2_v[pl.ds(c * _RCH, _RCH)]
        pltpu.async_copy(rbuf, xs_hbm.at[i1], sem).wait()
        pltpu.async_copy(rbuf, xs_hbm.at[i2], sem).wait()


@functools.lru_cache(maxsize=None)
def _make_dispatch():
    return pl.kernel(
        _dispatch_body,
        out_type=[
            jax.ShapeDtypeStruct((_PAD, _D), jnp.float32),
            jax.ShapeDtypeStruct((_PAD, 128), jnp.float32),
        ],
        mesh=plsc.VectorSubcoreMesh(core_axis_name="c", subcore_axis_name="s"),
        scratch_types=[
            pltpu.VMEM((_TPW,), jnp.int32),
            pltpu.VMEM((_TPW,), jnp.int32),
            pltpu.VMEM((_TPW, 128), jnp.float32),
            pltpu.VMEM((_RCH, _D), jnp.float32),
            pltpu.SemaphoreType.DMA,
        ],
    )


def _ffn_body(bem_ref, nv_ref, xs_ref, gs_ref, w1_ref, w3_ref, w2_ref, ys_ref):
    b = pl.program_id(0)
    f = pl.program_id(1)

    @pl.when(nv_ref[b] > 0)
    def _():
        x = xs_ref[...]
        a = jnp.dot(x, w1_ref[0], preferred_element_type=jnp.float32)
        g = jnp.dot(x, w3_ref[0], preferred_element_type=jnp.float32)
        h = a * jax.nn.sigmoid(a) * g
        y = jnp.dot(h, w2_ref[0], preferred_element_type=jnp.float32)

        @pl.when(f == 0)
        def _():
            ys_ref[...] = y

        @pl.when(f > 0)
        def _():
            ys_ref[...] = ys_ref[...] + y

        @pl.when(f == _NF - 1)
        def _():
            ys_ref[...] = ys_ref[...] * gs_ref[:, 0:1]


def _ffn(bem, nv, xs, gs, w1, w3, w2):
    grid_spec = pltpu.PrefetchScalarGridSpec(
        num_scalar_prefetch=2,
        grid=(_NB, _NF),
        in_specs=[
            pl.BlockSpec((_BLK, _D), lambda b, f, bem, nv: (b, 0)),
            pl.BlockSpec((_BLK, 128), lambda b, f, bem, nv: (b, 0)),
            pl.BlockSpec((1, _D, _FT), lambda b, f, bem, nv: (bem[b], 0, f)),
            pl.BlockSpec((1, _D, _FT), lambda b, f, bem, nv: (bem[b], 0, f)),
            pl.BlockSpec((1, _FT, _D), lambda b, f, bem, nv: (bem[b], f, 0)),
        ],
        out_specs=pl.BlockSpec((_BLK, _D), lambda b, f, bem, nv: (b, 0)),
    )
    return pl.pallas_call(
        _ffn_body,
        grid_spec=grid_spec,
        out_shape=jax.ShapeDtypeStruct((_PAD, _D), jnp.float32),
        compiler_params=pltpu.CompilerParams(
            dimension_semantics=("arbitrary", "arbitrary")),
    )(bem, nv, xs, gs, w1, w3, w2)


def _combine_body(ys_hbm, p1_hbm, p2_hbm, out_hbm, p1_v, p2_v, buf_a, buf_b,
                  sem_a, sem_b):
    wid = lax.axis_index("s") * _NC + lax.axis_index("c")
    base = wid * _TPW
    pltpu.sync_copy(p1_hbm.at[pl.ds(base, _TPW)], p1_v)
    pltpu.sync_copy(p2_hbm.at[pl.ds(base, _TPW)], p2_v)
    for c in range(_TPW // _RCH):
        i1 = p1_v[pl.ds(c * _RCH, _RCH)]
        i2 = p2_v[pl.ds(c * _RCH, _RCH)]
        pltpu.async_copy(ys_hbm.at[i1], buf_a, sem_a).wait()
        pltpu.async_copy(ys_hbm.at[i2], buf_b, sem_b).wait()
        for i in range(_RCH):
            def _add(j, carry, i=i):
                buf_a[i, pl.ds(j * 16, 16)] = (buf_a[i, pl.ds(j * 16, 16)] +
                                               buf_b[i, pl.ds(j * 16, 16)])
                return carry
            lax.fori_loop(0, _D // 16, _add, 0)
        pltpu.sync_copy(buf_a, out_hbm.at[pl.ds(base + c * _RCH, _RCH)])


@functools.lru_cache(maxsize=None)
def _make_combine():
    return pl.kernel(
        _combine_body,
        out_type=jax.ShapeDtypeStruct((_T, _D), jnp.float32),
        mesh=plsc.VectorSubcoreMesh(core_axis_name="c", subcore_axis_name="s"),
        scratch_types=[
            pltpu.VMEM((_TPW,), jnp.int32),
            pltpu.VMEM((_TPW,), jnp.int32),
            pltpu.VMEM((_RCH, _D), jnp.float32),
            pltpu.VMEM((_RCH, _D), jnp.float32),
            pltpu.SemaphoreType.DMA,
            pltpu.SemaphoreType.DMA,
        ],
    )


def kernel(data, w_router, w1, w2, w3):
    logits, p1, p2, g1x, g2x, bem, nv = _router(data, w_router)
    p1f = p1.reshape(_T)
    p2f = p2.reshape(_T)
    xs, gs = _make_dispatch()(data, p1f, p2f, g1x, g2x)
    ys = _ffn(bem.reshape(_NB), nv.reshape(_NB), xs, gs, w1, w3, w2)
    out = _make_combine()(ys, p1f, p2f)
    return out, logits
